# baseline (speedup 1.0000x reference)
import jax
import jax.numpy as jnp
from jax import lax
from jax.experimental import pallas as pl
from jax.experimental.pallas import tpu as pltpu
from jax.experimental.pallas import tpu_sc as plsc

B, N, D_IN, D_HID = 8192, 50, 10, 24
D_MLP, D_OUT = 36, 11
L = 16
NC, NS = 2, 16
NW = NC * NS
CHUNKS = B // L
CPW = CHUNKS // NW
SPW = B // NW

_P_WGAT = 0
_P_CSRC = _P_WGAT + D_IN * D_HID
_P_CDST = _P_CSRC + D_IN
_P_END = _P_CDST + D_IN
_P_PAD = ((_P_END + 15) // 16) * 16

_Q_W1 = 0
_Q_B1 = _Q_W1 + D_HID * D_MLP
_Q_W2 = _Q_B1 + D_MLP
_Q_B2 = _Q_W2 + D_MLP * D_OUT
_Q_END = _Q_B2 + D_OUT
_Q_PAD = ((_Q_END + 15) // 16) * 16


def _leaky(x):
    return jnp.where(x >= 0, x, 0.2 * x)


def _body(obs_hbm, par_hbm, mlp_hbm, out_hbm, obs_v, h_v, s_v, d_v, pool_v,
          hid_v, out_v, mlp_v, par_sh, par_v):
    sid = lax.axis_index("s")
    wid = sid * NC + lax.axis_index("c")

    @pl.when(sid == 0)
    def _stage_params():
        pltpu.sync_copy(par_hbm, par_sh)

    plsc.subcore_barrier()
    pltpu.sync_copy(par_sh, par_v)
    pltpu.sync_copy(mlp_hbm, mlp_v)
    zero = jnp.zeros((L,), jnp.float32)

    def chunk_body(c, carry):
        pltpu.sync_copy(obs_hbm.at[wid * CPW + c], obs_v)

        def n_body(n, dmax):
            ov = [obs_v[n, dd, :] for dd in range(D_IN)]
            for hh in range(D_HID):
                acc = ov[0] * par_v[_P_WGAT + hh]
                for dd in range(1, D_IN):
                    acc = acc + ov[dd] * par_v[_P_WGAT + dd * D_HID + hh]
                h_v[n, hh, :] = acc
            s = ov[0] * par_v[_P_CSRC]
            d = ov[0] * par_v[_P_CDST]
            for dd in range(1, D_IN):
                s = s + ov[dd] * par_v[_P_CSRC + dd]
                d = d + ov[dd] * par_v[_P_CDST + dd]
            s_v[n, :] = s
            d_v[n, :] = d
            return jnp.maximum(dmax, d)

        dmax = lax.fori_loop(0, N, n_body, jnp.full((L,), -jnp.inf,
                                                    jnp.float32))

        for hh in range(D_HID):
            pool_v[hh, :] = zero

        def i_body(i, carry2):
            x = s_v[i, :]
            m = _leaky(x + dmax)

            def j_body(j, zacc):
                z, accs = zacc
                p = jnp.exp(_leaky(x + d_v[j, :]) - m)
                new = tuple(accs[hh] + p * h_v[j, hh, :]
                            for hh in range(D_HID))
                return (z + p, new)

            z, accs = lax.fori_loop(
                0, N, j_body, (zero, tuple(zero for _ in range(D_HID))))
            for hh in range(D_HID):
                o = accs[hh] / z
                eo = jnp.where(o >= 0, o, jnp.exp(o) - 1.0)
                pool_v[hh, :] = pool_v[hh, :] + eo
            return carry2

        lax.fori_loop(0, N, i_body, 0)

        pooled = [pool_v[kk, :] * jnp.float32(1.0 / N) for kk in range(D_HID)]
        for mm in range(D_MLP):
            acc = pooled[0] * mlp_v[_Q_W1 + mm, :]
            for kk in range(1, D_HID):
                acc = acc + pooled[kk] * mlp_v[_Q_W1 + kk * D_MLP + mm, :]
            acc = acc + mlp_v[_Q_B1 + mm, :]
            hid_v[mm, :] = jnp.maximum(acc, 0.0)

        for oo in range(D_OUT):
            acc = hid_v[0, :] * mlp_v[_Q_W2 + oo, :]
            for mm in range(1, D_MLP):
                acc = acc + hid_v[mm, :] * mlp_v[_Q_W2 + mm * D_OUT + oo, :]
            acc = acc + mlp_v[_Q_B2 + oo, :]
            out_v[oo, pl.ds(c * L, L)] = acc
        return carry

    lax.fori_loop(0, CPW, chunk_body, 0)
    pltpu.sync_copy(out_v, out_hbm.at[wid])


_sc_call = pl.kernel(
    _body,
    mesh=plsc.VectorSubcoreMesh(core_axis_name="c", subcore_axis_name="s"),
    compiler_params=pltpu.CompilerParams(use_tc_tiling_on_sc=False),
    out_type=jax.ShapeDtypeStruct((NW, D_OUT, SPW), jnp.float32),
    scratch_types=[
        pltpu.VMEM((N, D_IN, L), jnp.float32),
        pltpu.VMEM((N, D_HID, L), jnp.float32),
        pltpu.VMEM((N, L), jnp.float32),
        pltpu.VMEM((N, L), jnp.float32),
        pltpu.VMEM((D_HID, L), jnp.float32),
        pltpu.VMEM((D_MLP, L), jnp.float32),
        pltpu.VMEM((D_OUT, SPW), jnp.float32),
        pltpu.VMEM((_Q_PAD, L), jnp.float32),
        pltpu.VMEM_SHARED((_P_PAD,), jnp.float32),
        pltpu.SMEM((_P_PAD,), jnp.float32),
    ],
)


@jax.jit
def kernel(obs, W_gat, a_src, a_dst, W1, b1, W2, b2):
    c_src = W_gat @ a_src
    c_dst = W_gat @ a_dst
    params = jnp.concatenate([
        W_gat.reshape(-1), c_src, c_dst,
        jnp.zeros((_P_PAD - _P_END,), jnp.float32),
    ])
    mlp = jnp.concatenate([
        W1.reshape(-1), b1, W2.reshape(-1), b2,
        jnp.zeros((_Q_PAD - _Q_END,), jnp.float32),
    ])
    mlp_bc = jnp.broadcast_to(mlp[:, None], (_Q_PAD, L))
    obs_r = obs.reshape(CHUNKS, L, N, D_IN).transpose(0, 2, 3, 1)
    out = _sc_call(obs_r, params, mlp_bc)
    return out.transpose(0, 2, 1).reshape(B, D_OUT)
